# baseline (device time: 75824 ns/iter reference)
import jax
import jax.numpy as jnp
from jax import lax
from jax.experimental import pallas as pl
from jax.experimental.pallas import tpu as pltpu

N_DEV = 4
N_GLOBAL = 8192
EPS = 1e-5
ROWS_C = 48
BLK = 4


def _fused_body(x_ref, gamma_ref, out_ref, comm_ref, scale_ref,
                send_sems, recv_sems):
    b = pl.program_id(0)
    me = lax.axis_index("i")

    @pl.when(b == 0)
    def _():
        x = x_ref[...]
        p = jnp.sum(x * x, axis=2)

        barrier = pltpu.get_barrier_semaphore()
        for k in range(1, N_DEV):
            peer = (me + k) % N_DEV
            pl.semaphore_signal(
                barrier, inc=1,
                device_id=(peer,), device_id_type=pl.DeviceIdType.MESH,
            )
        pl.semaphore_wait(barrier, N_DEV - 1)

        comm_ref[me] = p

        sends = []
        for k in range(1, N_DEV):
            peer = (me + k) % N_DEV
            rdma = pltpu.make_async_remote_copy(
                src_ref=comm_ref.at[me],
                dst_ref=comm_ref.at[me],
                send_sem=send_sems.at[k - 1],
                recv_sem=recv_sems.at[me],
                device_id=(peer,),
                device_id_type=pl.DeviceIdType.MESH,
            )
            rdma.start()
            sends.append(rdma)

        for k in range(1, N_DEV):
            peer = (me + k) % N_DEV
            recv = pltpu.make_async_remote_copy(
                src_ref=comm_ref.at[peer],
                dst_ref=comm_ref.at[peer],
                send_sem=send_sems.at[k - 1],
                recv_sem=recv_sems.at[peer],
                device_id=(peer,),
                device_id_type=pl.DeviceIdType.MESH,
            )
            recv.wait_recv()
        for s in sends:
            s.wait_send()

        total = comm_ref[0] + comm_ref[1] + comm_ref[2] + comm_ref[3]
        scale_ref[...] = lax.rsqrt(total * (1.0 / N_GLOBAL) + EPS)

    g = gamma_ref[...][None, :, :]
    s = scale_ref[pl.ds(BLK * b, BLK), :][:, :, None]
    out_ref[...] = x_ref[pl.ds(BLK * b, BLK)] * g * s


def kernel(x, gamma):
    m, n_local = x.shape
    x3 = x.reshape(ROWS_C, 128, n_local)
    gamma2 = gamma.reshape(1, n_local)
    n_blocks = ROWS_C // BLK

    out3 = pl.pallas_call(
        _fused_body,
        grid=(n_blocks,),
        in_specs=[
            pl.BlockSpec(memory_space=pltpu.VMEM),
            pl.BlockSpec(memory_space=pltpu.VMEM),
        ],
        out_specs=pl.BlockSpec((BLK, 128, n_local), lambda b: (b, 0, 0)),
        out_shape=jax.ShapeDtypeStruct((ROWS_C, 128, n_local), jnp.float32),
        scratch_shapes=[
            pltpu.VMEM((N_DEV, ROWS_C, 128), jnp.float32),
            pltpu.VMEM((ROWS_C, 128), jnp.float32),
            pltpu.SemaphoreType.DMA((N_DEV - 1,)),
            pltpu.SemaphoreType.DMA((N_DEV,)),
        ],
        compiler_params=pltpu.CompilerParams(
            collective_id=0,
            vmem_limit_bytes=100 * 1024 * 1024,
        ),
    )(x3, gamma2)

    return out3.reshape(m, n_local)
